# baseline (speedup 1.0000x reference)
import functools

import jax
import jax.numpy as jnp
from jax import lax
from jax.experimental import pallas as pl
from jax.experimental.pallas import tpu as pltpu
from jax.experimental.pallas import tpu_sc as plsc

_S = 128
_BJ = 512
_NPAD = 8


def _sc_compact_body(n, cp_hbm, emb_hbm, compact_hbm, ncp_hbm, counts_hbm,
                     cp_v, idx_v, stage_v, counts_v, inidx_v, outidx_v,
                     rows_v, sem_g, sem_s):
    i32 = jnp.int32
    c = lax.axis_index("c")
    s = lax.axis_index("s")
    b = c * 2 + s // 8
    rank = s % 8
    wid32 = c * 16 + s
    chunk = n // 8
    nrow = n + _NPAD

    base_rows = b * n + rank * chunk
    pltpu.sync_copy(cp_hbm.at[pl.ds(base_rows, chunk)], cp_v)

    lanes = lax.iota(i32, 16)
    off = jnp.int32(0)
    for i in range(chunk // 16):
        cpv = cp_v[pl.ds(i * 16, 16)]
        m = cpv != 0
        gi = base_rows + i * 16 + lanes
        pref = plsc.cumsum(m.astype(i32))
        pos = jnp.where(m, off + pref - 1, chunk)
        plsc.store_scatter(idx_v, [pos], gi)
        off = off + pref[15]
    local_cnt = off

    stage_v[...] = jnp.where(lanes == 0, local_cnt, 0)
    pltpu.sync_copy(stage_v, counts_hbm.at[wid32])
    plsc.subcore_barrier()

    row0 = c * 16 + (s // 8) * 8
    pltpu.sync_copy(counts_hbm.at[pl.ds(row0, 8)], counts_v)
    base = jnp.int32(0)
    ncp_b = jnp.int32(0)
    for j in range(8):
        cj = counts_v[j][0]
        base = base + jnp.where(j < rank, cj, 0)
        ncp_b = ncp_b + cj

    @pl.when(rank == 0)
    def _():
        stage_v[...] = jnp.where(lanes == 0, ncp_b, 0)
        pltpu.sync_copy(stage_v, ncp_hbm.at[b])

    dump = b * nrow + n
    out0 = b * nrow + base
    for g in range(chunk // 128):
        for v in range(8):
            o = g * 128 + v * 16
            kvec = o + lanes
            valid = kvec < local_cnt
            srcidx = jnp.where(valid, idx_v[pl.ds(o, 16)], 0)
            dstidx = jnp.where(valid, out0 + kvec, dump)
            inidx_v[pl.ds(v * 16, 16)] = srcidx
            outidx_v[pl.ds(v * 16, 16)] = dstidx
        pltpu.async_copy(emb_hbm.at[inidx_v], rows_v, sem_g).wait()
        pltpu.async_copy(rows_v, compact_hbm.at[outidx_v], sem_s).wait()


def _sc_compact(cp_flat, emb2d, B, N, D):
    f32 = jnp.float32
    i32 = jnp.int32
    mesh = plsc.VectorSubcoreMesh(core_axis_name="c", subcore_axis_name="s")
    return pl.kernel(
        functools.partial(_sc_compact_body, N),
        compiler_params=pltpu.CompilerParams(
            use_tc_tiling_on_sc=False,
            needs_layout_passes=False,
        ),
        out_type=(
            jax.ShapeDtypeStruct((B * (N + _NPAD), D), f32),
            jax.ShapeDtypeStruct((B, 16), i32),
            jax.ShapeDtypeStruct((32, 16), i32),
        ),
        mesh=mesh,
        scratch_types=[
            pltpu.VMEM((N // 8,), i32),
            pltpu.VMEM((N // 8 + 16,), i32),
            pltpu.VMEM((16,), i32),
            pltpu.VMEM((8, 16), i32),
            pltpu.VMEM((128,), i32),
            pltpu.VMEM((128,), i32),
            pltpu.VMEM((128, D), f32),
            pltpu.SemaphoreType.DMA,
            pltpu.SemaphoreType.DMA,
        ],
    )(cp_flat, emb2d)


def _softplus(x):
    return jnp.maximum(x, 0.0) + jnp.log(1.0 + jnp.exp(-jnp.abs(x)))


def _tc_body(beta_ref, e_ref, sid_ref, cp_ref, cE_ref, ncp_ref,
             out_ref, acc_ref, *, n):
    b = pl.program_id(0)

    @pl.when(b == 0)
    def _init():
        acc_ref[0] = 0.0
        acc_ref[1] = 0.0

    f32 = jnp.float32
    i32 = jnp.int32
    E = e_ref[0]
    bb = beta_ref[0]
    w = cp_ref[0]
    sid = sid_ref[0]
    ncp = ncp_ref[b, 0]
    ncp_f = ncp.astype(f32)

    n_non = n - ncp_f
    pos = jnp.sum(_softplus(-bb) * w) / jnp.maximum(ncp_f, 1.0)
    neg_sum = jnp.sum(_softplus(bb) * (1.0 - w))
    neg = jnp.where(n_non > 0, neg_sum / jnp.maximum(n_non, 1.0), 0.0)
    beta_loss = pos + 0.5 * neg

    seg_i = lax.broadcasted_iota(i32, (_S, n), 0)
    ohT = (seg_i == sid).astype(f32)
    cnt = jnp.sum(ohT, axis=1, keepdims=True)
    dn = (((1,), (0,)), ((), ()))
    sum_e = lax.dot_general(ohT, E, dn, preferred_element_type=f32)
    E2 = E * E
    sq_col = jnp.sum(E2, axis=1, keepdims=True)
    sumsq = lax.dot_general(ohT, sq_col, dn, preferred_element_type=f32)

    idx_row = lax.broadcasted_iota(i32, (1, n), 1)
    cand = jnp.where(w > 0, idx_row, n)
    m = jnp.where(ohT > 0, cand, n)
    first_cp = jnp.min(m, axis=1, keepdims=True)
    seg_j = lax.broadcasted_iota(i32, (_S, n), 1)
    ohF = (seg_j == first_cp).astype(f32)
    C = lax.dot_general(ohF, E, dn, preferred_element_type=f32)
    dot_cs = jnp.sum(C * sum_e, axis=1, keepdims=True)
    csq = jnp.sum(C * C, axis=1, keepdims=True)
    inst_mean = (sumsq - 2.0 * dot_cs + cnt * csq) / jnp.maximum(cnt, 1.0)
    use = (cnt > 0) & (first_cp < n)
    attraction = jnp.sum(jnp.where(use, inst_mean, 0.0))

    dn_bt = (((1,), (1,)), ((), ()))
    nblk = (ncp + _BJ - 1) // _BJ

    def jb_body(jb, rep):
        j0 = jb * _BJ
        Ej = cE_ref[0, pl.ds(j0, _BJ), :]
        Ej2 = Ej * Ej
        sqj_col = jnp.sum(Ej2, axis=1, keepdims=True)
        colmask = (j0 + lax.broadcasted_iota(i32, (1, _BJ), 1)) < ncp
        rowmask = (j0 + lax.broadcasted_iota(i32, (_BJ, 1), 0)) < ncp
        sqj_row = lax.dot_general(jnp.ones((1, Ej.shape[1]), f32), Ej2,
                                  dn_bt, preferred_element_type=f32)
        Gd = lax.dot_general(Ej, Ej, dn_bt, preferred_element_type=f32)
        argd = 2.0 * Gd - sqj_col - sqj_row
        argd = jnp.where(colmask & rowmask, argd, -1e30)
        rep = rep + jnp.sum(jnp.exp(argd))

        def ib_body(ib, rep2):
            Ei = cE_ref[0, pl.ds(ib * _BJ, _BJ), :]
            Ei2 = Ei * Ei
            sqi_col = jnp.sum(Ei2, axis=1, keepdims=True)
            G = lax.dot_general(Ei, Ej, dn_bt, preferred_element_type=f32)
            arg = 2.0 * G - sqi_col - sqj_row
            arg = jnp.where(colmask, arg, -1e30)
            return rep2 + 2.0 * jnp.sum(jnp.exp(arg))

        return lax.fori_loop(0, jb, ib_body, rep)

    rep = lax.fori_loop(0, nblk, jb_body, jnp.float32(0.0))
    rep_mean = rep / jnp.maximum(ncp_f * ncp_f, 1.0)
    repulsion = jnp.where(ncp_f > 1, rep_mean, 0.0)

    active = ncp_f > 0
    contrib = beta_loss + attraction + repulsion
    acc_ref[0] += jnp.where(active, contrib, 0.0)
    acc_ref[1] += jnp.where(active, 1.0, 0.0)
    total = acc_ref[0]
    countf = acc_ref[1]
    loss = jnp.where(countf > 0.0, total / jnp.maximum(countf, 1.0), 0.0)
    out_ref[...] = jnp.broadcast_to(loss, (1, 1))


def kernel(beta, embed, slice_id, is_cp):
    B, N, D = embed.shape
    f32 = jnp.float32
    beta_row = beta[..., 0].astype(f32).reshape(B, 1, N)
    cp_row = is_cp.astype(f32).reshape(B, 1, N)
    sid_row = slice_id.astype(jnp.int32).reshape(B, 1, N)

    cp_flat = is_cp.astype(jnp.int32).reshape(B * N)
    emb2d = embed.reshape(B * N, D)
    compact, ncp_tbl, _counts = _sc_compact(cp_flat, emb2d, B, N, D)
    compact3d = compact.reshape(B, N + _NPAD, D)

    out = pl.pallas_call(
        functools.partial(_tc_body, n=N),
        grid=(B,),
        in_specs=[
            pl.BlockSpec((1, 1, N), lambda b: (b, 0, 0)),
            pl.BlockSpec((1, N, D), lambda b: (b, 0, 0)),
            pl.BlockSpec((1, 1, N), lambda b: (b, 0, 0)),
            pl.BlockSpec((1, 1, N), lambda b: (b, 0, 0)),
            pl.BlockSpec((1, N + _NPAD, D), lambda b: (b, 0, 0)),
            pl.BlockSpec(memory_space=pltpu.SMEM),
        ],
        out_specs=pl.BlockSpec((1, 1), lambda b: (0, 0)),
        out_shape=jax.ShapeDtypeStruct((1, 1), f32),
        scratch_shapes=[pltpu.SMEM((2,), f32)],
    )(beta_row, embed, sid_row, cp_row, compact3d, ncp_tbl)
    return out[0, 0]


# speedup vs baseline: 1.0887x; 1.0887x over previous
import functools

import jax
import jax.numpy as jnp
from jax import lax
from jax.experimental import pallas as pl
from jax.experimental.pallas import tpu as pltpu
from jax.experimental.pallas import tpu_sc as plsc

_S = 128
_BJ = 512
_NPAD = 8


def _sc_compact_body(n, cp_hbm, emb_hbm, compact_hbm, ncp_hbm, counts_hbm,
                     cp_v, idx_v, stage_v, counts_v,
                     inidx0, inidx1, inidx2, inidx3,
                     outidx0, outidx1, outidx2, outidx3,
                     rows0, rows1, rows2, rows3, sem_g, sem_s):
    inidx = (inidx0, inidx1, inidx2, inidx3)
    outidx = (outidx0, outidx1, outidx2, outidx3)
    rows = (rows0, rows1, rows2, rows3)
    i32 = jnp.int32
    c = lax.axis_index("c")
    s = lax.axis_index("s")
    b = c * 2 + s // 8
    rank = s % 8
    wid32 = c * 16 + s
    chunk = n // 8
    nrow = n + _NPAD

    base_rows = b * n + rank * chunk
    pltpu.sync_copy(cp_hbm.at[pl.ds(base_rows, chunk)], cp_v)

    lanes = lax.iota(i32, 16)
    off = jnp.int32(0)
    for i in range(chunk // 16):
        cpv = cp_v[pl.ds(i * 16, 16)]
        m = cpv != 0
        gi = base_rows + i * 16 + lanes
        pref = plsc.cumsum(m.astype(i32))
        pos = jnp.where(m, off + pref - 1, chunk)
        plsc.store_scatter(idx_v, [pos], gi)
        off = off + pref[15]
    local_cnt = off

    ngrp = chunk // 128
    gathers = []
    for g in range(ngrp):
        for v in range(8):
            o = g * 128 + v * 16
            kvec = o + lanes
            valid = kvec < local_cnt
            inidx[g][pl.ds(v * 16, 16)] = jnp.where(
                valid, idx_v[pl.ds(o, 16)], 0)
    for g in range(ngrp):
        gathers.append(pltpu.async_copy(
            emb_hbm.at[inidx[g]], rows[g], sem_g))

    stage_v[...] = jnp.where(lanes == 0, local_cnt, 0)
    pltpu.sync_copy(stage_v, counts_hbm.at[wid32])
    plsc.subcore_barrier()

    row0 = c * 16 + (s // 8) * 8
    pltpu.sync_copy(counts_hbm.at[pl.ds(row0, 8)], counts_v)
    base = jnp.int32(0)
    ncp_b = jnp.int32(0)
    for j in range(8):
        cj = counts_v[j][0]
        base = base + jnp.where(j < rank, cj, 0)
        ncp_b = ncp_b + cj

    @pl.when(rank == 0)
    def _():
        stage_v[...] = jnp.where(lanes == 0, ncp_b, 0)
        pltpu.sync_copy(stage_v, ncp_hbm.at[b])

    dump = b * nrow + n
    out0 = b * nrow + base
    scatters = []
    for g in range(ngrp):
        for v in range(8):
            o = g * 128 + v * 16
            kvec = o + lanes
            valid = kvec < local_cnt
            outidx[g][pl.ds(v * 16, 16)] = jnp.where(valid, out0 + kvec, dump)
        gathers[g].wait()
        scatters.append(pltpu.async_copy(
            rows[g], compact_hbm.at[outidx[g]], sem_s))
    for cp_desc in scatters:
        cp_desc.wait()


def _sc_compact(cp_flat, emb2d, B, N, D):
    f32 = jnp.float32
    i32 = jnp.int32
    mesh = plsc.VectorSubcoreMesh(core_axis_name="c", subcore_axis_name="s")
    return pl.kernel(
        functools.partial(_sc_compact_body, N),
        compiler_params=pltpu.CompilerParams(
            use_tc_tiling_on_sc=False,
            needs_layout_passes=False,
        ),
        out_type=(
            jax.ShapeDtypeStruct((B * (N + _NPAD), D), f32),
            jax.ShapeDtypeStruct((B, 16), i32),
            jax.ShapeDtypeStruct((32, 16), i32),
        ),
        mesh=mesh,
        scratch_types=(
            [
                pltpu.VMEM((N // 8,), i32),
                pltpu.VMEM((N // 8 + 16,), i32),
                pltpu.VMEM((16,), i32),
                pltpu.VMEM((8, 16), i32),
            ]
            + [pltpu.VMEM((128,), i32) for _ in range(8)]
            + [pltpu.VMEM((128, D), f32) for _ in range(4)]
            + [pltpu.SemaphoreType.DMA, pltpu.SemaphoreType.DMA]
        ),
    )(cp_flat, emb2d)


def _softplus(x):
    return jnp.maximum(x, 0.0) + jnp.log(1.0 + jnp.exp(-jnp.abs(x)))


def _tc_body(beta_ref, e_ref, sid_ref, cp_ref, cE_ref, ncp_ref,
             out_ref, acc_ref, rep_ref, *, n):
    b = pl.program_id(0)

    @pl.when(b == 0)
    def _init():
        acc_ref[0] = 0.0
        acc_ref[1] = 0.0

    f32 = jnp.float32
    i32 = jnp.int32
    E = e_ref[0]
    bb = beta_ref[0]
    w = cp_ref[0]
    sid = sid_ref[0]
    ncp = ncp_ref[b, 0]
    ncp_f = ncp.astype(f32)

    n_non = n - ncp_f
    pos = jnp.sum(_softplus(-bb) * w) / jnp.maximum(ncp_f, 1.0)
    neg_sum = jnp.sum(_softplus(bb) * (1.0 - w))
    neg = jnp.where(n_non > 0, neg_sum / jnp.maximum(n_non, 1.0), 0.0)
    beta_loss = pos + 0.5 * neg

    seg_i = lax.broadcasted_iota(i32, (_S, n), 0)
    ohT = (seg_i == sid).astype(f32)
    cnt = jnp.sum(ohT, axis=1, keepdims=True)
    dn = (((1,), (0,)), ((), ()))
    sum_e = lax.dot_general(ohT, E, dn, preferred_element_type=f32)
    E2 = E * E
    sq_col = jnp.sum(E2, axis=1, keepdims=True)
    sumsq = lax.dot_general(ohT, sq_col, dn, preferred_element_type=f32)

    idx_row = lax.broadcasted_iota(i32, (1, n), 1)
    cand = jnp.where(w > 0, idx_row, n)
    m = jnp.where(ohT > 0, cand, n)
    first_cp = jnp.min(m, axis=1, keepdims=True)
    seg_j = lax.broadcasted_iota(i32, (_S, n), 1)
    ohF = (seg_j == first_cp).astype(f32)
    C = lax.dot_general(ohF, E, dn, preferred_element_type=f32)
    dot_cs = jnp.sum(C * sum_e, axis=1, keepdims=True)
    csq = jnp.sum(C * C, axis=1, keepdims=True)
    inst_mean = (sumsq - 2.0 * dot_cs + cnt * csq) / jnp.maximum(cnt, 1.0)
    use = (cnt > 0) & (first_cp < n)
    attraction = jnp.sum(jnp.where(use, inst_mean, 0.0))

    dn_bt = (((1,), (1,)), ((), ()))
    cE = cE_ref[0]
    rep_ref[0] = 0.0
    nb = n // _BJ
    for jb in range(nb):
        j0 = jb * _BJ

        @pl.when(j0 < ncp)
        def _(jb=jb, j0=j0):
            Ej = cE[j0:j0 + _BJ, :]
            Ej2 = Ej * Ej
            sqj_col = jnp.sum(Ej2, axis=1, keepdims=True)
            colmask = (j0 + lax.broadcasted_iota(i32, (1, _BJ), 1)) < ncp
            rowmask = (j0 + lax.broadcasted_iota(i32, (_BJ, 1), 0)) < ncp
            sqj_row = lax.dot_general(jnp.ones((1, cE.shape[1]), f32), Ej2,
                                      dn_bt, preferred_element_type=f32)
            Gd = lax.dot_general(Ej, Ej, dn_bt, preferred_element_type=f32)
            argd = 2.0 * Gd - sqj_col - sqj_row
            argd = jnp.where(colmask & rowmask, argd, -1e30)
            acc = jnp.sum(jnp.exp(argd))
            if jb > 0:
                Ei = cE[:j0, :]
                Ei2 = Ei * Ei
                sqi_col = jnp.sum(Ei2, axis=1, keepdims=True)
                G = lax.dot_general(Ei, Ej, dn_bt, preferred_element_type=f32)
                arg = 2.0 * G - sqi_col - sqj_row
                arg = jnp.where(colmask, arg, -1e30)
                acc = acc + 2.0 * jnp.sum(jnp.exp(arg))
            rep_ref[0] += acc

    rep = rep_ref[0]
    rep_mean = rep / jnp.maximum(ncp_f * ncp_f, 1.0)
    repulsion = jnp.where(ncp_f > 1, rep_mean, 0.0)

    active = ncp_f > 0
    contrib = beta_loss + attraction + repulsion
    acc_ref[0] += jnp.where(active, contrib, 0.0)
    acc_ref[1] += jnp.where(active, 1.0, 0.0)
    total = acc_ref[0]
    countf = acc_ref[1]
    loss = jnp.where(countf > 0.0, total / jnp.maximum(countf, 1.0), 0.0)
    out_ref[...] = jnp.broadcast_to(loss, (1, 1))


def kernel(beta, embed, slice_id, is_cp):
    B, N, D = embed.shape
    f32 = jnp.float32
    beta_row = beta[..., 0].astype(f32).reshape(B, 1, N)
    cp_row = is_cp.astype(f32).reshape(B, 1, N)
    sid_row = slice_id.astype(jnp.int32).reshape(B, 1, N)

    cp_flat = is_cp.astype(jnp.int32).reshape(B * N)
    emb2d = embed.reshape(B * N, D)
    compact, ncp_tbl, _counts = _sc_compact(cp_flat, emb2d, B, N, D)
    compact3d = compact.reshape(B, N + _NPAD, D)

    out = pl.pallas_call(
        functools.partial(_tc_body, n=N),
        grid=(B,),
        in_specs=[
            pl.BlockSpec((1, 1, N), lambda b: (b, 0, 0)),
            pl.BlockSpec((1, N, D), lambda b: (b, 0, 0)),
            pl.BlockSpec((1, 1, N), lambda b: (b, 0, 0)),
            pl.BlockSpec((1, 1, N), lambda b: (b, 0, 0)),
            pl.BlockSpec((1, N + _NPAD, D), lambda b: (b, 0, 0)),
            pl.BlockSpec(memory_space=pltpu.SMEM),
        ],
        out_specs=pl.BlockSpec((1, 1), lambda b: (0, 0)),
        out_shape=jax.ShapeDtypeStruct((1, 1), f32),
        scratch_shapes=[pltpu.SMEM((2,), f32), pltpu.SMEM((1,), f32)],
    )(beta_row, embed, sid_row, cp_row, compact3d, ncp_tbl)
    return out[0, 0]


# speedup vs baseline: 1.4816x; 1.3609x over previous
import functools

import jax
import jax.numpy as jnp
from jax import lax
from jax.experimental import pallas as pl
from jax.experimental.pallas import tpu as pltpu
from jax.experimental.pallas import tpu_sc as plsc

_S = 128
_BJ = 512
_NPAD = 8


def _sc_compact_body(n, cp_hbm, embf_hbm, compact_hbm, ncp_hbm, counts_hbm,
                     cp_v, idx_v, emb_v, crow_v, stage_v, counts_v,
                     outidx0, outidx1, outidx2, outidx3, sem_s):
    i32 = jnp.int32
    outidx = (outidx0, outidx1, outidx2, outidx3)
    c = lax.axis_index("c")
    s = lax.axis_index("s")
    b = c * 2 + s // 8
    rank = s % 8
    wid32 = c * 16 + s
    chunk = n // 8
    nrow = n + _NPAD

    base_rows = b * n + rank * chunk
    pltpu.sync_copy(cp_hbm.at[pl.ds(base_rows, chunk)], cp_v)
    pltpu.sync_copy(embf_hbm.at[pl.ds(base_rows, chunk)], emb_v)

    lanes = lax.iota(i32, 16)
    off = jnp.int32(0)
    for i in range(chunk // 16):
        cpv = cp_v[pl.ds(i * 16, 16)]
        m = cpv != 0
        li = i * 16 + lanes
        pref = plsc.cumsum(m.astype(i32))
        pos = jnp.where(m, off + pref - 1, chunk)
        plsc.store_scatter(idx_v, [pos], li)
        off = off + pref[15]
    local_cnt = off

    stage_v[...] = jnp.where(lanes == 0, local_cnt, 0)
    pltpu.sync_copy(stage_v, counts_hbm.at[wid32])

    def pack_one(k, _):
        lid = idx_v[pl.ds(k, 16)][0]
        row = plsc.load_gather(emb_v, [jnp.broadcast_to(lid, (16,)), lanes])
        plsc.store_scatter(crow_v, [jnp.broadcast_to(k, (16,)), lanes], row)
        return 0
    lax.fori_loop(0, local_cnt, pack_one, 0)

    plsc.subcore_barrier()
    row0 = c * 16 + (s // 8) * 8
    pltpu.sync_copy(counts_hbm.at[pl.ds(row0, 8)], counts_v)
    base = jnp.int32(0)
    ncp_b = jnp.int32(0)
    for j in range(8):
        cj = counts_v[j][0]
        base = base + jnp.where(j < rank, cj, 0)
        ncp_b = ncp_b + cj

    @pl.when(rank == 0)
    def _():
        stage_v[...] = jnp.where(lanes == 0, ncp_b, 0)
        pltpu.sync_copy(stage_v, ncp_hbm.at[b])

    dump = b * nrow + n
    out0 = b * nrow + base
    scatters = []
    for g in range(chunk // 128):
        for v in range(8):
            o = g * 128 + v * 16
            kvec = o + lanes
            valid = kvec < local_cnt
            outidx[g][pl.ds(v * 16, 16)] = jnp.where(valid, out0 + kvec, dump)
        scatters.append(pltpu.async_copy(
            crow_v.at[pl.ds(g * 128, 128), :],
            compact_hbm.at[outidx[g]], sem_s))
    for cp_desc in scatters:
        cp_desc.wait()


def _sc_compact(cp_flat, emb_flat, B, N, D):
    f32 = jnp.float32
    i32 = jnp.int32
    mesh = plsc.VectorSubcoreMesh(core_axis_name="c", subcore_axis_name="s")
    chunk = N // 8
    return pl.kernel(
        functools.partial(_sc_compact_body, N),
        compiler_params=pltpu.CompilerParams(
            use_tc_tiling_on_sc=False,
            needs_layout_passes=False,
        ),
        out_type=(
            jax.ShapeDtypeStruct((B * (N + _NPAD), D), f32),
            jax.ShapeDtypeStruct((B, 16), i32),
            jax.ShapeDtypeStruct((32, 16), i32),
        ),
        mesh=mesh,
        scratch_types=(
            [
                pltpu.VMEM((chunk,), i32),
                pltpu.VMEM((chunk + 16,), i32),
                pltpu.VMEM((chunk, D), f32),
                pltpu.VMEM((chunk, D), f32),
                pltpu.VMEM((16,), i32),
                pltpu.VMEM((8, 16), i32),
            ]
            + [pltpu.VMEM((128,), i32) for _ in range(4)]
            + [pltpu.SemaphoreType.DMA]
        ),
    )(cp_flat, emb_flat)


def _softplus(x):
    return jnp.maximum(x, 0.0) + jnp.log(1.0 + jnp.exp(-jnp.abs(x)))


def _tc1_body(beta_ref, e_ref, sid_ref, cp_ref, out_ref, *, n):
    b = pl.program_id(0)
    f32 = jnp.float32
    i32 = jnp.int32
    E = e_ref[0]
    bb = beta_ref[0]
    w = cp_ref[0]
    sid = sid_ref[0]

    n_cp = jnp.sum(w)
    n_non = n - n_cp
    pos = jnp.sum(_softplus(-bb) * w) / jnp.maximum(n_cp, 1.0)
    neg_sum = jnp.sum(_softplus(bb) * (1.0 - w))
    neg = jnp.where(n_non > 0, neg_sum / jnp.maximum(n_non, 1.0), 0.0)
    beta_loss = pos + 0.5 * neg

    seg_i = lax.broadcasted_iota(i32, (_S, n), 0)
    ohT = (seg_i == sid).astype(f32)
    cnt = jnp.sum(ohT, axis=1, keepdims=True)
    dn = (((1,), (0,)), ((), ()))
    sum_e = lax.dot_general(ohT, E, dn, preferred_element_type=f32)
    E2 = E * E
    sq_col = jnp.sum(E2, axis=1, keepdims=True)
    sumsq = lax.dot_general(ohT, sq_col, dn, preferred_element_type=f32)

    idx_row = lax.broadcasted_iota(i32, (1, n), 1)
    cand = jnp.where(w > 0, idx_row, n)
    m = jnp.where(ohT > 0, cand, n)
    first_cp = jnp.min(m, axis=1, keepdims=True)
    seg_j = lax.broadcasted_iota(i32, (_S, n), 1)
    ohF = (seg_j == first_cp).astype(f32)
    C = lax.dot_general(ohF, E, dn, preferred_element_type=f32)
    dot_cs = jnp.sum(C * sum_e, axis=1, keepdims=True)
    csq = jnp.sum(C * C, axis=1, keepdims=True)
    inst_mean = (sumsq - 2.0 * dot_cs + cnt * csq) / jnp.maximum(cnt, 1.0)
    use = (cnt > 0) & (first_cp < n)
    attraction = jnp.sum(jnp.where(use, inst_mean, 0.0))

    out_ref[...] = jnp.broadcast_to(beta_loss + attraction, (1, 1, 1))


def _tc2_body(cE_ref, ncp_ref, out_ref, *, n):
    b = pl.program_id(0)
    f32 = jnp.float32
    i32 = jnp.int32
    ncp = ncp_ref[b, 0]
    ncp_f = ncp.astype(f32)
    dn_bt = (((1,), (1,)), ((), ()))
    cE = cE_ref[0]
    out_ref[...] = jnp.zeros((1, 1, 1), f32)
    for jb in range(n // _BJ):
        j0 = jb * _BJ

        @pl.when(j0 < ncp)
        def _(jb=jb, j0=j0):
            Ej = cE[j0:j0 + _BJ, :]
            Ej2 = Ej * Ej
            sqj_col = jnp.sum(Ej2, axis=1, keepdims=True)
            colmask = (j0 + lax.broadcasted_iota(i32, (1, _BJ), 1)) < ncp
            rowmask = (j0 + lax.broadcasted_iota(i32, (_BJ, 1), 0)) < ncp
            sqj_row = lax.dot_general(jnp.ones((1, cE.shape[1]), f32), Ej2,
                                      dn_bt, preferred_element_type=f32)
            Gd = lax.dot_general(Ej, Ej, dn_bt, preferred_element_type=f32)
            argd = 2.0 * Gd - sqj_col - sqj_row
            argd = jnp.where(colmask & rowmask, argd, -1e30)
            acc = jnp.sum(jnp.exp(argd))
            if jb > 0:
                Ei = cE[:j0, :]
                Ei2 = Ei * Ei
                sqi_col = jnp.sum(Ei2, axis=1, keepdims=True)
                G = lax.dot_general(Ei, Ej, dn_bt, preferred_element_type=f32)
                arg = 2.0 * G - sqi_col - sqj_row
                arg = jnp.where(colmask, arg, -1e30)
                acc = acc + 2.0 * jnp.sum(jnp.exp(arg))
            out_ref[...] = out_ref[...] + jnp.broadcast_to(acc, (1, 1, 1))

    rep = out_ref[...][0, 0, 0]
    rep_mean = rep / jnp.maximum(ncp_f * ncp_f, 1.0)
    out_ref[...] = jnp.broadcast_to(
        jnp.where(ncp_f > 1, rep_mean, 0.0), (1, 1, 1))


def kernel(beta, embed, slice_id, is_cp):
    B, N, D = embed.shape
    f32 = jnp.float32
    beta_row = beta[..., 0].astype(f32).reshape(B, 1, N)
    cp_row = is_cp.astype(f32).reshape(B, 1, N)
    sid_row = slice_id.astype(jnp.int32).reshape(B, 1, N)

    cp_flat = is_cp.astype(jnp.int32).reshape(B * N)
    emb2d = embed.reshape(B * N, D)
    compact, ncp_tbl, _counts = _sc_compact(cp_flat, emb2d, B, N, D)
    compact3d = compact.reshape(B, N + _NPAD, D)

    ba = pl.pallas_call(
        functools.partial(_tc1_body, n=N),
        grid=(B,),
        in_specs=[
            pl.BlockSpec((1, 1, N), lambda b: (b, 0, 0)),
            pl.BlockSpec((1, N, D), lambda b: (b, 0, 0)),
            pl.BlockSpec((1, 1, N), lambda b: (b, 0, 0)),
            pl.BlockSpec((1, 1, N), lambda b: (b, 0, 0)),
        ],
        out_specs=pl.BlockSpec((1, 1, 1), lambda b: (b, 0, 0)),
        out_shape=jax.ShapeDtypeStruct((B, 1, 1), f32),
    )(beta_row, embed, sid_row, cp_row)

    rep = pl.pallas_call(
        functools.partial(_tc2_body, n=N),
        grid=(B,),
        in_specs=[
            pl.BlockSpec((1, N + _NPAD, D), lambda b: (b, 0, 0)),
            pl.BlockSpec(memory_space=pltpu.SMEM),
        ],
        out_specs=pl.BlockSpec((1, 1, 1), lambda b: (b, 0, 0)),
        out_shape=jax.ShapeDtypeStruct((B, 1, 1), f32),
    )(compact3d, ncp_tbl)

    ncp_b = ncp_tbl[:, 0].astype(f32)
    active = ncp_b > 0
    contrib = ba[:, 0, 0] + rep[:, 0, 0]
    total = jnp.sum(jnp.where(active, contrib, 0.0))
    count = jnp.sum(active.astype(f32))
    return jnp.where(count > 0, total / jnp.maximum(count, 1.0),
                     jnp.float32(0.0))
